# baseline (speedup 1.0000x reference)
import functools

import jax
import jax.numpy as jnp
from jax import lax
from jax.experimental import pallas as pl
from jax.experimental.pallas import tpu as pltpu

_THRESH = 0.6


def _nms_chunk_kernel(nb, c,
                      x1r, y1r, x2r, y2r, ar,
                      sup0, supr):
    supr[...] = sup0[...]

    row_i = lax.broadcasted_iota(jnp.int32, (c, c), 0)
    col_i = lax.broadcasted_iota(jnp.int32, (c, c), 1)
    tri = (col_i > row_i)

    def to_col(v):
        return jnp.transpose(v, (1, 0))

    def iou_ge(cx1, cy1, cx2, cy2, ca, rx1, ry1, rx2, ry2, ra):
        xx1 = jnp.maximum(cx1, rx1)
        yy1 = jnp.maximum(cy1, ry1)
        xx2 = jnp.minimum(cx2, rx2)
        yy2 = jnp.minimum(cy2, ry2)
        w = jnp.maximum(0.0, xx2 - xx1 + 1.0)
        h = jnp.maximum(0.0, yy2 - yy1 + 1.0)
        inter = w * h
        union = ca + ra - inter
        return (inter / union) >= _THRESH

    def chunk_step(ci, _):
        rx1 = x1r[ci]
        ry1 = y1r[ci]
        rx2 = x2r[ci]
        ry2 = y2r[ci]
        ra = ar[ci]

        cx1 = to_col(rx1)
        cy1 = to_col(ry1)
        cx2 = to_col(rx2)
        cy2 = to_col(ry2)
        ca = to_col(ra)

        hit = iou_ge(cx1, cy1, cx2, cy2, ca, rx1, ry1, rx2, ry2, ra)
        m = jnp.where(hit & tri, 1.0, 0.0)

        s0 = supr[ci]

        def fix_cond(carry):
            _, changed = carry
            return changed

        def fix_body(carry):
            s, _ = carry
            active = 1.0 - s
            t = lax.dot_general(active, m, (((1,), (0,)), ((), ())),
                                preferred_element_type=jnp.float32)
            s_new = jnp.maximum(s0, jnp.where(t > 0.0, 1.0, 0.0))
            return s_new, jnp.any(s_new != s)

        s_fin, _ = lax.while_loop(fix_cond, fix_body, (s0, True))
        supr[ci] = s_fin
        keep_row = 1.0 - s_fin

        def build(cj):
            ox1 = x1r[cj]
            oy1 = y1r[cj]
            ox2 = x2r[cj]
            oy2 = y2r[cj]
            oa = ar[cj]
            return jnp.where(
                iou_ge(cx1, cy1, cx2, cy2, ca, ox1, oy1, ox2, oy2, oa),
                1.0, 0.0)

        def suppress(d):
            t = lax.dot_general(keep_row, d, (((1,), (0,)), ((), ())),
                                preferred_element_type=jnp.float32)
            return jnp.where(t > 0.0, 1.0, 0.0)

        cstart = jnp.minimum(ci + 1, nb - 1)
        d0 = build(cstart)

        def cross(cj, d_prev):
            supr[cj - 1] = jnp.maximum(supr[cj - 1], suppress(d_prev))
            return build(cj)

        d_last = lax.fori_loop(cstart + 1, nb, cross, d0, unroll=False)
        guard = jnp.where(ci + 1 < nb, 1.0, 0.0)
        supr[nb - 1] = jnp.maximum(supr[nb - 1], suppress(d_last) * guard)
        return 0

    lax.fori_loop(0, nb, chunk_step, 0, unroll=False)


def _run_nms_sorted(x1s, y1s, x2s, y2s, areas_s, chunk):
    n = x1s.shape[0]
    c = chunk
    nb = -(-n // c)
    npad = nb * c
    pad = npad - n

    def prep(v):
        return jnp.pad(v, (0, pad)).reshape(nb, 1, c)

    x1r = prep(x1s)
    y1r = prep(y1s)
    x2r = prep(x2s)
    y2r = prep(y2s)
    arr = prep(areas_s)
    sup0 = jnp.pad(jnp.zeros((n,), jnp.float32), (0, pad),
                   constant_values=1.0).reshape(nb, 1, c)

    sup = pl.pallas_call(
        functools.partial(_nms_chunk_kernel, nb, c),
        out_shape=jax.ShapeDtypeStruct((nb, 1, c), jnp.float32),
    )(x1r, y1r, x2r, y2r, arr, sup0)
    return sup.reshape(npad)[:n]


def kernel(boxes, scores):
    n = boxes.shape[0]
    x1 = boxes[:, 0]
    y1 = boxes[:, 1]
    x2 = boxes[:, 2]
    y2 = boxes[:, 3]
    areas = (x2 - x1 + 1.0) * (y2 - y1 + 1.0)
    order = jnp.argsort(-scores)

    x1s = x1[order]
    y1s = y1[order]
    x2s = x2[order]
    y2s = y2[order]
    areas_s = areas[order]

    sup = _run_nms_sorted(x1s, y1s, x2s, y2s, areas_s, chunk=1024)

    keep_sorted = sup < 0.5
    keep = jnp.zeros((n,), dtype=bool).at[order].set(keep_sorted)
    keep_f = keep.astype(boxes.dtype)
    labels = jnp.zeros((n, 1), dtype=boxes.dtype)
    bbox_pred = jnp.concatenate([labels, scores[:, None], boxes],
                                axis=1) * keep_f[:, None]
    bbox_num = jnp.sum(keep).astype(jnp.int32)[None]
    nms_keep_idx = jnp.nonzero(keep, size=n, fill_value=0)[0]
    return bbox_pred, bbox_num, nms_keep_idx


# speedup vs baseline: 1.5375x; 1.5375x over previous
import functools

import jax
import jax.numpy as jnp
from jax import lax
from jax.experimental import pallas as pl
from jax.experimental.pallas import tpu as pltpu

_THRESH = 0.6


def _nms_chunk_kernel(nb, c,
                      x1r, y1r, x2r, y2r, ar,
                      sup0, supr):
    supr[...] = sup0[...]

    row_i = lax.broadcasted_iota(jnp.int32, (c, c), 0)
    col_i = lax.broadcasted_iota(jnp.int32, (c, c), 1)
    tri = (col_i > row_i)

    def to_col(v):
        return jnp.transpose(v, (1, 0))

    def iou_ge(cx1, cy1, cx2, cy2, ca, rx1, ry1, rx2, ry2, ra):
        xx1 = jnp.maximum(cx1, rx1)
        yy1 = jnp.maximum(cy1, ry1)
        xx2 = jnp.minimum(cx2, rx2)
        yy2 = jnp.minimum(cy2, ry2)
        w = jnp.maximum(0.0, xx2 - xx1 + 1.0)
        h = jnp.maximum(0.0, yy2 - yy1 + 1.0)
        inter = w * h
        union = ca + ra - inter
        return (inter / union) >= _THRESH

    def chunk_step(ci, _):
        rx1 = x1r[ci]
        ry1 = y1r[ci]
        rx2 = x2r[ci]
        ry2 = y2r[ci]
        ra = ar[ci]

        cx1 = to_col(rx1)
        cy1 = to_col(ry1)
        cx2 = to_col(rx2)
        cy2 = to_col(ry2)
        ca = to_col(ra)

        hit = iou_ge(cx1, cy1, cx2, cy2, ca, rx1, ry1, rx2, ry2, ra)
        m = jnp.where(hit & tri, 1.0, 0.0)

        s0 = supr[ci]

        def fix_cond(carry):
            _, changed = carry
            return changed

        def fix_body(carry):
            s, _ = carry
            active = 1.0 - s
            t = lax.dot_general(active, m, (((1,), (0,)), ((), ())),
                                preferred_element_type=jnp.float32)
            s_new = jnp.maximum(s0, jnp.where(t > 0.0, 1.0, 0.0))
            return s_new, jnp.any(s_new != s)

        s_fin, _ = lax.while_loop(fix_cond, fix_body, (s0, True))
        supr[ci] = s_fin
        keep_row = 1.0 - s_fin

        def build(cj):
            ox1 = x1r[cj]
            oy1 = y1r[cj]
            ox2 = x2r[cj]
            oy2 = y2r[cj]
            oa = ar[cj]
            return jnp.where(
                iou_ge(cx1, cy1, cx2, cy2, ca, ox1, oy1, ox2, oy2, oa),
                1.0, 0.0)

        def suppress(d):
            t = lax.dot_general(keep_row, d, (((1,), (0,)), ((), ())),
                                preferred_element_type=jnp.float32)
            return jnp.where(t > 0.0, 1.0, 0.0)

        ntarget = nb - (ci + 1)

        def cross2(k, _):
            cj0 = ci + 1 + 2 * k
            cj1 = cj0 + 1
            cjc0 = jnp.minimum(cj0, nb - 1)
            cjc1 = jnp.minimum(cj1, nb - 1)
            g0 = jnp.where(cj0 < nb, 1.0, 0.0)
            g1 = jnp.where(cj1 < nb, 1.0, 0.0)
            t0 = suppress(build(cjc0)) * g0
            t1 = suppress(build(cjc1)) * g1
            supr[cjc0] = jnp.maximum(supr[cjc0], t0)
            supr[cjc1] = jnp.maximum(supr[cjc1], t1)
            return 0

        lax.fori_loop(0, (ntarget + 1) // 2, cross2, 0, unroll=False)
        return 0

    lax.fori_loop(0, nb, chunk_step, 0, unroll=False)


def _run_nms_sorted(x1s, y1s, x2s, y2s, areas_s, chunk):
    n = x1s.shape[0]
    c = chunk
    nb = -(-n // c)
    npad = nb * c
    pad = npad - n

    def prep(v):
        return jnp.pad(v, (0, pad)).reshape(nb, 1, c)

    x1r = prep(x1s)
    y1r = prep(y1s)
    x2r = prep(x2s)
    y2r = prep(y2s)
    arr = prep(areas_s)
    sup0 = jnp.pad(jnp.zeros((n,), jnp.float32), (0, pad),
                   constant_values=1.0).reshape(nb, 1, c)

    sup = pl.pallas_call(
        functools.partial(_nms_chunk_kernel, nb, c),
        out_shape=jax.ShapeDtypeStruct((nb, 1, c), jnp.float32),
    )(x1r, y1r, x2r, y2r, arr, sup0)
    return sup.reshape(npad)[:n]


def kernel(boxes, scores):
    n = boxes.shape[0]
    x1 = boxes[:, 0]
    y1 = boxes[:, 1]
    x2 = boxes[:, 2]
    y2 = boxes[:, 3]
    areas = (x2 - x1 + 1.0) * (y2 - y1 + 1.0)
    order = jnp.argsort(-scores)

    x1s = x1[order]
    y1s = y1[order]
    x2s = x2[order]
    y2s = y2[order]
    areas_s = areas[order]

    sup = _run_nms_sorted(x1s, y1s, x2s, y2s, areas_s, chunk=1024)

    keep_sorted = sup < 0.5
    keep = jnp.zeros((n,), dtype=bool).at[order].set(keep_sorted)
    keep_f = keep.astype(boxes.dtype)
    labels = jnp.zeros((n, 1), dtype=boxes.dtype)
    bbox_pred = jnp.concatenate([labels, scores[:, None], boxes],
                                axis=1) * keep_f[:, None]
    bbox_num = jnp.sum(keep).astype(jnp.int32)[None]
    nms_keep_idx = jnp.nonzero(keep, size=n, fill_value=0)[0]
    return bbox_pred, bbox_num, nms_keep_idx


# speedup vs baseline: 1.8268x; 1.1882x over previous
import functools

import jax
import jax.numpy as jnp
from jax import lax
from jax.experimental import pallas as pl
from jax.experimental.pallas import tpu as pltpu
from jax.experimental.pallas import tpu_sc as plsc

_THRESH = 0.6


def _nms_chunk_kernel(nb, c,
                      x1r, y1r, x2r, y2r, ar,
                      sup0, supr):
    supr[...] = sup0[...]

    row_i = lax.broadcasted_iota(jnp.int32, (c, c), 0)
    col_i = lax.broadcasted_iota(jnp.int32, (c, c), 1)
    tri = (col_i > row_i)

    def to_col(v):
        return jnp.transpose(v, (1, 0))

    def iou_ge(cx1, cy1, cx2, cy2, ca, rx1, ry1, rx2, ry2, ra):
        xx1 = jnp.maximum(cx1, rx1)
        yy1 = jnp.maximum(cy1, ry1)
        xx2 = jnp.minimum(cx2, rx2)
        yy2 = jnp.minimum(cy2, ry2)
        w = jnp.maximum(0.0, xx2 - xx1 + 1.0)
        h = jnp.maximum(0.0, yy2 - yy1 + 1.0)
        inter = w * h
        union = ca + ra - inter
        return (inter / union) >= _THRESH

    def chunk_step(ci, _):
        rx1 = x1r[ci]
        ry1 = y1r[ci]
        rx2 = x2r[ci]
        ry2 = y2r[ci]
        ra = ar[ci]

        cx1 = to_col(rx1)
        cy1 = to_col(ry1)
        cx2 = to_col(rx2)
        cy2 = to_col(ry2)
        ca = to_col(ra)

        hit = iou_ge(cx1, cy1, cx2, cy2, ca, rx1, ry1, rx2, ry2, ra)
        m = jnp.where(hit & tri, 1.0, 0.0)

        s0 = supr[ci]

        def fix_cond(carry):
            _, changed = carry
            return changed

        def fix_body(carry):
            s, _ = carry
            active = 1.0 - s
            t = lax.dot_general(active, m, (((1,), (0,)), ((), ())),
                                preferred_element_type=jnp.float32)
            s_new = jnp.maximum(s0, jnp.where(t > 0.0, 1.0, 0.0))
            return s_new, jnp.any(s_new != s)

        s_fin, _ = lax.while_loop(fix_cond, fix_body, (s0, True))
        supr[ci] = s_fin
        keep_row = 1.0 - s_fin

        def build(cj):
            ox1 = x1r[cj]
            oy1 = y1r[cj]
            ox2 = x2r[cj]
            oy2 = y2r[cj]
            oa = ar[cj]
            return jnp.where(
                iou_ge(cx1, cy1, cx2, cy2, ca, ox1, oy1, ox2, oy2, oa),
                1.0, 0.0)

        def suppress(d):
            t = lax.dot_general(keep_row, d, (((1,), (0,)), ((), ())),
                                preferred_element_type=jnp.float32)
            return jnp.where(t > 0.0, 1.0, 0.0)

        def cross(cj, _):
            t = suppress(build(cj))
            supr[cj] = jnp.maximum(supr[cj], t)
            return 0

        lax.fori_loop(ci + 1, nb, cross, 0, unroll=False)
        return 0

    lax.fori_loop(0, nb, chunk_step, 0, unroll=False)


def _run_nms_sorted(x1s, y1s, x2s, y2s, areas_s, chunk):
    n = x1s.shape[0]
    c = chunk
    nb = -(-n // c)
    npad = nb * c
    pad = npad - n

    def prep(v):
        return jnp.pad(v, (0, pad)).reshape(nb, 1, c)

    x1r = prep(x1s)
    y1r = prep(y1s)
    x2r = prep(x2s)
    y2r = prep(y2s)
    arr = prep(areas_s)
    sup0 = jnp.pad(jnp.zeros((n,), jnp.float32), (0, pad),
                   constant_values=1.0).reshape(nb, 1, c)

    sup = pl.pallas_call(
        functools.partial(_nms_chunk_kernel, nb, c),
        out_shape=jax.ShapeDtypeStruct((nb, 1, c), jnp.float32),
    )(x1r, y1r, x2r, y2r, arr, sup0)
    return sup.reshape(npad)[:n]


def _make_sc_tail(n):
    assert n % 16 == 0
    nchunk = n // 16
    mesh = plsc.VectorSubcoreMesh(core_axis_name="c", subcore_axis_name="s")

    def body(keep_s_hbm, order_hbm, keep_o_hbm, idx_hbm, cnt_hbm,
             ks_v, ord_v, ko_v, idx_v, cnt_v, sem):
        wid = lax.axis_index("s") * 2 + lax.axis_index("c")

        @pl.when(wid == 0)
        def _():
            pltpu.sync_copy(keep_s_hbm, ks_v)
            pltpu.sync_copy(order_hbm, ord_v)
            iota16 = lax.iota(jnp.int32, 16)
            z16 = jnp.zeros((16,), jnp.int32)

            def zero_step(i, carry):
                idx_v[pl.ds(i * 16, 16)] = z16
                return carry

            lax.fori_loop(0, nchunk, zero_step, 0)

            def scat_step(i, carry):
                k16 = ks_v[pl.ds(i * 16, 16)]
                o16 = ord_v[pl.ds(i * 16, 16)]
                plsc.store_scatter(ko_v, [o16], k16)
                return carry

            lax.fori_loop(0, nchunk, scat_step, 0)

            def comp_step(i, off):
                k16 = ko_v[pl.ds(i * 16, 16)]
                m = k16 != 0.0
                m32 = m.astype(jnp.int32)
                cum = plsc.cumsum(m32)
                pos = off + cum - 1
                ids = i * 16 + iota16
                plsc.store_scatter(idx_v, [pos], ids, mask=m)
                return off + jnp.sum(m32)

            total = lax.fori_loop(0, nchunk, comp_step, 0)
            cnt_v[...] = z16 + total
            pltpu.sync_copy(ko_v, keep_o_hbm)
            pltpu.sync_copy(idx_v, idx_hbm)
            pltpu.sync_copy(cnt_v, cnt_hbm)

    return pl.kernel(
        body,
        out_type=[jax.ShapeDtypeStruct((n,), jnp.float32),
                  jax.ShapeDtypeStruct((n,), jnp.int32),
                  jax.ShapeDtypeStruct((16,), jnp.int32)],
        mesh=mesh,
        compiler_params=pltpu.CompilerParams(needs_layout_passes=False),
        scratch_types=[pltpu.VMEM((n,), jnp.float32),
                       pltpu.VMEM((n,), jnp.int32),
                       pltpu.VMEM((n,), jnp.float32),
                       pltpu.VMEM((n,), jnp.int32),
                       pltpu.VMEM((16,), jnp.int32),
                       pltpu.SemaphoreType.DMA],
    )


def kernel(boxes, scores):
    n = boxes.shape[0]
    x1 = boxes[:, 0]
    y1 = boxes[:, 1]
    x2 = boxes[:, 2]
    y2 = boxes[:, 3]
    areas = (x2 - x1 + 1.0) * (y2 - y1 + 1.0)
    order = jnp.argsort(-scores)

    x1s = x1[order]
    y1s = y1[order]
    x2s = x2[order]
    y2s = y2[order]
    areas_s = areas[order]

    sup = _run_nms_sorted(x1s, y1s, x2s, y2s, areas_s, chunk=1024)

    keep_sorted = 1.0 - sup
    keep_f, nms_keep_idx, cnt = _make_sc_tail(n)(
        keep_sorted, order.astype(jnp.int32))
    labels = jnp.zeros((n, 1), dtype=boxes.dtype)
    bbox_pred = jnp.concatenate([labels, scores[:, None], boxes],
                                axis=1) * keep_f[:, None]
    bbox_num = cnt[:1]
    return bbox_pred, bbox_num, nms_keep_idx
